# baseline (speedup 1.0000x reference)
import functools

import jax
import jax.numpy as jnp
from jax import lax
from jax.experimental import pallas as pl
from jax.experimental.pallas import tpu as pltpu
from jax.experimental.pallas import tpu_sc as plsc

B = 16384
RT, RS, RP = 8, 12, 4
NC, NS, L = 2, 16, 16
NW = NC * NS
BW = B // NW
CHUNK = 128
NCH = BW // CHUNK
NG = BW // L


def _sc_body(tg, xg0, xg1, yg0, yg1, pg, xraw, yraw, uraw, corev,
             et, exv, eyv, euv, out,
             tg_v, xg0_v, xg1_v, yg0_v, yg1_v, pg_v,
             xraw_v, yraw_v, uraw_v,
             rows_t, x_dd, y_dd, rows_p,
             core_v, out_v, sem):
    wid = lax.axis_index("s") * NC + lax.axis_index("c")
    base = wid * BW

    pltpu.sync_copy(tg.at[wid], tg_v)
    pltpu.sync_copy(xg0.at[wid], xg0_v)
    pltpu.sync_copy(xg1.at[wid], xg1_v)
    pltpu.sync_copy(yg0.at[wid], yg0_v)
    pltpu.sync_copy(yg1.at[wid], yg1_v)
    pltpu.sync_copy(pg.at[wid], pg_v)
    pltpu.sync_copy(xraw.at[wid], xraw_v)
    pltpu.sync_copy(yraw.at[wid], yraw_v)
    pltpu.sync_copy(uraw.at[wid], uraw_v)
    pltpu.sync_copy(corev, core_v)

    descs = []
    for j in range(NCH):
        s = pl.ds(j * CHUNK, CHUNK)
        descs.append(pltpu.async_copy(et.at[tg_v.at[j]], rows_t.at[s], sem))
        descs.append(pltpu.async_copy(exv.at[xg0_v.at[j]], x_dd.at[0].at[s], sem))
        descs.append(pltpu.async_copy(exv.at[xg1_v.at[j]], x_dd.at[1].at[s], sem))
        descs.append(pltpu.async_copy(eyv.at[yg0_v.at[j]], y_dd.at[0].at[s], sem))
        descs.append(pltpu.async_copy(eyv.at[yg1_v.at[j]], y_dd.at[1].at[s], sem))
        descs.append(pltpu.async_copy(euv.at[pg_v.at[j]], rows_p.at[s], sem))
    for d in descs:
        d.wait()

    lane = lax.iota(jnp.int32, L)

    cvecs = [core_v[pl.ds(v * L, L)] for v in range(RT * RS * RP // L)]

    def cscal(n):
        return cvecs[n // L][n % L]

    def group(g, carry):
        row = lane + g * L
        sl = pl.ds(g * L, L)
        xo = (xraw_v[sl] & 1) * 4
        yo = (yraw_v[sl] & 1) * 4
        po = (uraw_v[sl] & 1) * 4
        tcols = [plsc.load_gather(rows_t, [row, jnp.full((L,), i, jnp.int32)])
                 for i in range(RT)]
        xcols = [plsc.load_gather(x_dd, [(xo + j) >> 3, row, (xo + j) & 7])
                 for j in range(RS)]
        ycols = [plsc.load_gather(y_dd, [(yo + j) >> 3, row, (yo + j) & 7])
                 for j in range(RS)]
        qcols = [plsc.load_gather(rows_p, [row, po + k]) for k in range(RP)]
        sxy = [xcols[j] * ycols[j] for j in range(RS)]
        acc = None
        for i in range(RT):
            e_i = None
            for j in range(RS):
                off = (i * RS + j) * RP
                m = cscal(off) * qcols[0]
                for k in range(1, RP):
                    m = m + cscal(off + k) * qcols[k]
                term = m * sxy[j]
                e_i = term if e_i is None else e_i + term
            contrib = tcols[i] * e_i
            acc = contrib if acc is None else acc + contrib
        out_v[sl] = acc
        return carry

    lax.fori_loop(0, NG, group, 0)
    pltpu.sync_copy(out_v, out.at[pl.ds(base, BW)])


def kernel(indices, core_tensor, edge_time, edge_space_x, edge_space_y,
           edge_physics):
    tu = indices[:, 0]
    xu = indices[:, 1]
    yu = indices[:, 2]
    uu = indices[:, 3]
    tg = tu.reshape(NW, NCH, CHUNK)
    xg0 = (xu + (xu >> 1)).reshape(NW, NCH, CHUNK)
    xg1 = xg0 + 1
    yg0 = (yu + (yu >> 1)).reshape(NW, NCH, CHUNK)
    yg1 = yg0 + 1
    pg = (uu >> 1).reshape(NW, NCH, CHUNK)
    xraw = xu.reshape(NW, BW)
    yraw = yu.reshape(NW, BW)
    uraw = uu.reshape(NW, BW)
    corev = core_tensor.reshape(RT * RS * RP)
    exv = edge_space_x.reshape(100000 * RS // 8, 8)
    eyv = edge_space_y.reshape(100000 * RS // 8, 8)
    euv = edge_physics.reshape(100000 * RP // 8, 8)

    mesh = plsc.VectorSubcoreMesh(core_axis_name="c", subcore_axis_name="s")
    call = functools.partial(
        pl.kernel,
        mesh=mesh,
        compiler_params=pltpu.CompilerParams(
            needs_layout_passes=False, use_tc_tiling_on_sc=False),
        out_type=jax.ShapeDtypeStruct((B,), jnp.float32),
        scratch_types=[
            pltpu.VMEM((NCH, CHUNK), jnp.int32),
            pltpu.VMEM((NCH, CHUNK), jnp.int32),
            pltpu.VMEM((NCH, CHUNK), jnp.int32),
            pltpu.VMEM((NCH, CHUNK), jnp.int32),
            pltpu.VMEM((NCH, CHUNK), jnp.int32),
            pltpu.VMEM((NCH, CHUNK), jnp.int32),
            pltpu.VMEM((BW,), jnp.int32),
            pltpu.VMEM((BW,), jnp.int32),
            pltpu.VMEM((BW,), jnp.int32),
            pltpu.VMEM((BW, 8), jnp.float32),
            pltpu.VMEM((2, BW, 8), jnp.float32),
            pltpu.VMEM((2, BW, 8), jnp.float32),
            pltpu.VMEM((BW, 8), jnp.float32),
            pltpu.VMEM((RT * RS * RP,), jnp.float32),
            pltpu.VMEM((BW,), jnp.float32),
            pltpu.SemaphoreType.DMA,
        ],
    )(_sc_body)
    return call(tg, xg0, xg1, yg0, yg1, pg, xraw, yraw, uraw, corev,
                edge_time, exv, eyv, euv)


# speedup vs baseline: 1.0882x; 1.0882x over previous
import functools

import jax
import jax.numpy as jnp
from jax import lax
from jax.experimental import pallas as pl
from jax.experimental.pallas import tpu as pltpu
from jax.experimental.pallas import tpu_sc as plsc

B = 16384
RT, RS, RP = 8, 12, 4
NC, NS, L = 2, 16, 16
NW = NC * NS
BW = B // NW
CHUNK = 128
NCH = BW // CHUNK
GPC = CHUNK // L
PD = 128


def _sc_body(tg, xg, yg, ug, corev, et, ex, ey, eu, out,
             tg_v, xg_v, yg_v, ug_v,
             pad_t, pad_x, pad_y, pad_u,
             core_v, out_v, sem):
    wid = lax.axis_index("s") * NC + lax.axis_index("c")
    base = wid * BW

    pltpu.sync_copy(tg.at[wid], tg_v)
    pltpu.sync_copy(xg.at[wid], xg_v)
    pltpu.sync_copy(yg.at[wid], yg_v)
    pltpu.sync_copy(ug.at[wid], ug_v)
    pltpu.sync_copy(corev, core_v)

    lane = lax.iota(jnp.int32, L)

    cvecs = [core_v[pl.ds(v * L, L)] for v in range(RT * RS * RP // L)]

    def cscal(n):
        return cvecs[n // L][n % L]

    def run_chunk(c):
        descs = [
            pltpu.async_copy(et.at[tg_v.at[c]], pad_t, sem),
            pltpu.async_copy(ex.at[xg_v.at[c]], pad_x, sem),
            pltpu.async_copy(ey.at[yg_v.at[c]], pad_y, sem),
            pltpu.async_copy(eu.at[ug_v.at[c]], pad_u, sem),
        ]
        for d in descs:
            d.wait()

        def group(g, carry):
            row = lane + g * L
            tcols = [plsc.load_gather(pad_t, [row, jnp.full((L,), i, jnp.int32)])
                     for i in range(RT)]
            xcols = [plsc.load_gather(pad_x, [row, jnp.full((L,), j, jnp.int32)])
                     for j in range(RS)]
            ycols = [plsc.load_gather(pad_y, [row, jnp.full((L,), j, jnp.int32)])
                     for j in range(RS)]
            qcols = [plsc.load_gather(pad_u, [row, jnp.full((L,), k, jnp.int32)])
                     for k in range(RP)]
            sxy = [xcols[j] * ycols[j] for j in range(RS)]
            acc = None
            for i in range(RT):
                e_i = None
                for j in range(RS):
                    off = (i * RS + j) * RP
                    m = cscal(off) * qcols[0]
                    for k in range(1, RP):
                        m = m + cscal(off + k) * qcols[k]
                    term = m * sxy[j]
                    e_i = term if e_i is None else e_i + term
                contrib = tcols[i] * e_i
                acc = contrib if acc is None else acc + contrib
            out_v[pl.ds(c * CHUNK + g * L, L)] = acc
            return carry

        lax.fori_loop(0, GPC, group, 0)

    for c in range(NCH):
        run_chunk(c)

    pltpu.sync_copy(out_v, out.at[pl.ds(base, BW)])


def kernel(indices, core_tensor, edge_time, edge_space_x, edge_space_y,
           edge_physics):
    tg = indices[:, 0].reshape(NW, NCH, CHUNK)
    xg = indices[:, 1].reshape(NW, NCH, CHUNK)
    yg = indices[:, 2].reshape(NW, NCH, CHUNK)
    ug = indices[:, 3].reshape(NW, NCH, CHUNK)
    corev = core_tensor.reshape(RT * RS * RP)
    etp = jnp.pad(edge_time, ((0, 0), (0, PD - RT)))
    exp_ = jnp.pad(edge_space_x, ((0, 0), (0, PD - RS)))
    eyp = jnp.pad(edge_space_y, ((0, 0), (0, PD - RS)))
    eup = jnp.pad(edge_physics, ((0, 0), (0, PD - RP)))

    mesh = plsc.VectorSubcoreMesh(core_axis_name="c", subcore_axis_name="s")
    call = functools.partial(
        pl.kernel,
        mesh=mesh,
        compiler_params=pltpu.CompilerParams(
            needs_layout_passes=False, use_tc_tiling_on_sc=False),
        out_type=jax.ShapeDtypeStruct((B,), jnp.float32),
        scratch_types=[
            pltpu.VMEM((NCH, CHUNK), jnp.int32),
            pltpu.VMEM((NCH, CHUNK), jnp.int32),
            pltpu.VMEM((NCH, CHUNK), jnp.int32),
            pltpu.VMEM((NCH, CHUNK), jnp.int32),
            pltpu.VMEM((CHUNK, PD), jnp.float32),
            pltpu.VMEM((CHUNK, PD), jnp.float32),
            pltpu.VMEM((CHUNK, PD), jnp.float32),
            pltpu.VMEM((CHUNK, PD), jnp.float32),
            pltpu.VMEM((RT * RS * RP,), jnp.float32),
            pltpu.VMEM((BW,), jnp.float32),
            pltpu.SemaphoreType.DMA,
        ],
    )(_sc_body)
    return call(tg, xg, yg, ug, corev, etp, exp_, eyp, eup)


# speedup vs baseline: 1.0887x; 1.0004x over previous
import functools

import jax
import jax.numpy as jnp
from jax import lax
from jax.experimental import pallas as pl
from jax.experimental.pallas import tpu as pltpu
from jax.experimental.pallas import tpu_sc as plsc

B = 16384
RT, RS, RP = 8, 12, 4
NC, NS, L = 2, 16, 16
NW = NC * NS
BW = B // NW
CHUNK = 128
NCH = BW // CHUNK
GPC = CHUNK // L
PD = 128


def _sc_body(tg, xg, yg, ug, corev, et, ex, ey, eu, out,
             tg_v, xg_v, yg_v, ug_v,
             pad_t, pad_x, pad_y, pad_u,
             core_v, out_v, sem):
    wid = lax.axis_index("s") * NC + lax.axis_index("c")
    base = wid * BW

    pltpu.sync_copy(tg.at[pl.ds(base, BW)], tg_v)
    pltpu.sync_copy(xg.at[pl.ds(base, BW)], xg_v)
    pltpu.sync_copy(yg.at[pl.ds(base, BW)], yg_v)
    pltpu.sync_copy(ug.at[pl.ds(base, BW)], ug_v)
    pltpu.sync_copy(corev, core_v)

    lane = lax.iota(jnp.int32, L)

    cvecs = [core_v[pl.ds(v * L, L)] for v in range(RT * RS * RP // L)]

    def cscal(n):
        return cvecs[n // L][n % L]

    def run_chunk(c):
        s = pl.ds(c * CHUNK, CHUNK)
        descs = [
            pltpu.async_copy(et.at[tg_v.at[s]], pad_t, sem),
            pltpu.async_copy(ex.at[xg_v.at[s]], pad_x, sem),
            pltpu.async_copy(ey.at[yg_v.at[s]], pad_y, sem),
            pltpu.async_copy(eu.at[ug_v.at[s]], pad_u, sem),
        ]
        for d in descs:
            d.wait()

        def group(g, carry):
            row = lane + g * L
            tcols = [plsc.load_gather(pad_t, [row, jnp.full((L,), i, jnp.int32)])
                     for i in range(RT)]
            xcols = [plsc.load_gather(pad_x, [row, jnp.full((L,), j, jnp.int32)])
                     for j in range(RS)]
            ycols = [plsc.load_gather(pad_y, [row, jnp.full((L,), j, jnp.int32)])
                     for j in range(RS)]
            qcols = [plsc.load_gather(pad_u, [row, jnp.full((L,), k, jnp.int32)])
                     for k in range(RP)]
            sxy = [xcols[j] * ycols[j] for j in range(RS)]
            acc = None
            for i in range(RT):
                e_i = None
                for j in range(RS):
                    off = (i * RS + j) * RP
                    m = cscal(off) * qcols[0]
                    for k in range(1, RP):
                        m = m + cscal(off + k) * qcols[k]
                    term = m * sxy[j]
                    e_i = term if e_i is None else e_i + term
                contrib = tcols[i] * e_i
                acc = contrib if acc is None else acc + contrib
            out_v[pl.ds(c * CHUNK + g * L, L)] = acc
            return carry

        lax.fori_loop(0, GPC, group, 0)

    for c in range(NCH):
        run_chunk(c)

    pltpu.sync_copy(out_v, out.at[pl.ds(base, BW)])


def kernel(indices, core_tensor, edge_time, edge_space_x, edge_space_y,
           edge_physics):
    tg = indices[:, 0]
    xg = indices[:, 1]
    yg = indices[:, 2]
    ug = indices[:, 3]
    corev = core_tensor.reshape(RT * RS * RP)
    etp = jnp.pad(edge_time, ((0, 0), (0, PD - RT)))
    exp_ = jnp.pad(edge_space_x, ((0, 0), (0, PD - RS)))
    eyp = jnp.pad(edge_space_y, ((0, 0), (0, PD - RS)))
    eup = jnp.pad(edge_physics, ((0, 0), (0, PD - RP)))

    mesh = plsc.VectorSubcoreMesh(core_axis_name="c", subcore_axis_name="s")
    call = functools.partial(
        pl.kernel,
        mesh=mesh,
        compiler_params=pltpu.CompilerParams(
            needs_layout_passes=False, use_tc_tiling_on_sc=True),
        out_type=jax.ShapeDtypeStruct((B,), jnp.float32),
        scratch_types=[
            pltpu.VMEM((BW,), jnp.int32),
            pltpu.VMEM((BW,), jnp.int32),
            pltpu.VMEM((BW,), jnp.int32),
            pltpu.VMEM((BW,), jnp.int32),
            pltpu.VMEM((CHUNK, PD), jnp.float32),
            pltpu.VMEM((CHUNK, PD), jnp.float32),
            pltpu.VMEM((CHUNK, PD), jnp.float32),
            pltpu.VMEM((CHUNK, PD), jnp.float32),
            pltpu.VMEM((RT * RS * RP,), jnp.float32),
            pltpu.VMEM((BW,), jnp.float32),
            pltpu.SemaphoreType.DMA,
        ],
    )(_sc_body)
    return call(tg, xg, yg, ug, corev, etp, exp_, eyp, eup)
